# Initial kernel scaffold; baseline (speedup 1.0000x reference)
import jax
import jax.numpy as jnp
from jax.experimental import pallas as pl


def kernel(edge_index, r_ij, v, W1, b1, W2, b2, W3, b3):
    raise NotImplementedError("write your pallas kernel here")


# speedup vs baseline: 6.7488x; 6.7488x over previous
import jax
import jax.numpy as jnp
from jax import lax
from jax.experimental import pallas as pl
from jax.experimental.pallas import tpu as pltpu
from jax.experimental.pallas import tpu_sc as plsc

E = 6_400_000
NN = 100_000
L = 16
NW = 32
NCHUNK = 8
CHUNK = E // NCHUNK
BLK = 8_000
NBLK = CHUNK // BLK
TCB = 6_400
DB = 8_192

_mesh = plsc.VectorSubcoreMesh(core_axis_name="c", subcore_axis_name="s")


def _sc_gather_body(iarr, jarr, rflat, v0, v1, v2,
                    r2_out, d0_out, d1_out, d2_out,
                    tab, ibuf, jbuf, obuf):
    wid = lax.axis_index("s") * 2 + lax.axis_index("c")

    def v_task(vplane, dout, chunk):
        pltpu.sync_copy(vplane, tab)

        def blk(b, _):
            base = chunk * CHUNK + b * BLK
            pltpu.sync_copy(iarr.at[pl.ds(base, BLK)], ibuf)
            pltpu.sync_copy(jarr.at[pl.ds(base, BLK)], jbuf)

            def inner(k, _):
                sl = pl.ds(k * L, L)
                gi = plsc.load_gather(tab, [ibuf[sl]])
                gj = plsc.load_gather(tab, [jbuf[sl]])
                obuf[sl] = gi - gj
                return 0

            lax.fori_loop(0, BLK // L, inner, 0, unroll=4)
            pltpu.sync_copy(obuf, dout.at[pl.ds(base, BLK)])
            return 0

        lax.fori_loop(0, NBLK, blk, 0)

    for c, (vplane, dout) in enumerate(((v0, d0_out), (v1, d1_out),
                                        (v2, d2_out))):
        @pl.when(jnp.logical_and(wid >= c * NCHUNK, wid < (c + 1) * NCHUNK))
        def _(vplane=vplane, dout=dout, c=c):
            v_task(vplane, dout, wid - c * NCHUNK)

    @pl.when(wid >= 24)
    def _r_task():
        chunk = wid - 24
        sidx = lax.iota(jnp.int32, L) * 3

        def blk(b, _):
            base = chunk * CHUNK + b * BLK
            pltpu.sync_copy(rflat.at[pl.ds(base * 3, BLK * 3)],
                            tab.at[pl.ds(0, BLK * 3)])

            def inner(k, _):
                off = sidx + k * (3 * L)
                x = plsc.load_gather(tab, [off])
                y = plsc.load_gather(tab, [off + 1])
                z = plsc.load_gather(tab, [off + 2])
                obuf[pl.ds(k * L, L)] = x * x + y * y + z * z
                return 0

            lax.fori_loop(0, BLK // L, inner, 0, unroll=4)
            pltpu.sync_copy(obuf, r2_out.at[pl.ds(base, BLK)])
            return 0

        lax.fori_loop(0, NBLK, blk, 0)


_gather_call = pl.kernel(
    _sc_gather_body,
    out_type=tuple(jax.ShapeDtypeStruct((E,), jnp.float32) for _ in range(4)),
    mesh=_mesh,
    compiler_params=pltpu.CompilerParams(needs_layout_passes=False),
    scratch_types=[
        pltpu.VMEM((NN,), jnp.float32),
        pltpu.VMEM((BLK,), jnp.int32),
        pltpu.VMEM((BLK,), jnp.int32),
        pltpu.VMEM((BLK,), jnp.float32),
    ],
)


def _silu(x):
    return x * (1.0 / (1.0 + jnp.exp(-x)))


def _mlp_body(r2_ref, d0_ref, d1_ref, d2_ref,
              w1_ref, b1_ref, w2_ref, b2_ref, w3_ref, b3_ref,
              tf_ref, tb_ref):
    r = jnp.sqrt(r2_ref[...])
    w1 = w1_ref[...]
    a = w1[:, 0:1] * r + b1_ref[...]
    bb = (w1[:, 1:2] * d0_ref[...]
          + w1[:, 2:3] * d1_ref[...]
          + w1[:, 3:4] * d2_ref[...])
    h1f = _silu(a + bb)
    h1b = _silu(a - bb)
    w2 = w2_ref[...]
    h2f = _silu(jnp.dot(w2, h1f, preferred_element_type=jnp.float32) + b2_ref[...])
    h2b = _silu(jnp.dot(w2, h1b, preferred_element_type=jnp.float32) + b2_ref[...])
    w3c = w3_ref[...]
    tf_ref[...] = jnp.sum(w3c * h2f, axis=0, keepdims=True) + b3_ref[...]
    tb_ref[...] = jnp.sum(w3c * h2b, axis=0, keepdims=True) + b3_ref[...]


def _mlp_call(r2, d0, d1, d2, W1, b1c, W2, b2c, w3c, b3c):
    edge_spec = pl.BlockSpec((1, TCB), lambda b: (0, b))
    return pl.pallas_call(
        _mlp_body,
        grid=(E // TCB,),
        in_specs=[
            edge_spec, edge_spec, edge_spec, edge_spec,
            pl.BlockSpec((32, 4), lambda b: (0, 0)),
            pl.BlockSpec((32, 1), lambda b: (0, 0)),
            pl.BlockSpec((32, 32), lambda b: (0, 0)),
            pl.BlockSpec((32, 1), lambda b: (0, 0)),
            pl.BlockSpec((32, 1), lambda b: (0, 0)),
            pl.BlockSpec((1, 1), lambda b: (0, 0)),
        ],
        out_specs=(edge_spec, edge_spec),
        out_shape=(jax.ShapeDtypeStruct((1, E), jnp.float32),
                   jax.ShapeDtypeStruct((1, E), jnp.float32)),
    )(r2, d0, d1, d2, W1, b1c, W2, b2c, w3c, b3c)


def _sc_scatter_body(iarr, jarr, tf, tb, part_out, acc, ibuf, vbuf):
    wid = lax.axis_index("s") * 2 + lax.axis_index("c")
    kind = wid // NCHUNK
    chunk = wid % NCHUNK

    def z(k, _):
        acc[pl.ds(k * L, L)] = jnp.zeros((L,), jnp.float32)
        return 0

    lax.fori_loop(0, NN // L, z, 0, unroll=8)
    ones = jnp.ones((L,), jnp.float32)

    def scatter_task(idx_ref, val_ref, chunk):
        def blk(b, _):
            base = chunk * CHUNK + b * BLK
            pltpu.sync_copy(idx_ref.at[pl.ds(base, BLK)], ibuf)
            if val_ref is not None:
                pltpu.sync_copy(val_ref.at[pl.ds(base, BLK)], vbuf)

            def inner(k, _):
                sl = pl.ds(k * L, L)
                val = vbuf[sl] if val_ref is not None else ones
                plsc.addupdate_scatter(acc, [ibuf[sl]], val)
                return 0

            lax.fori_loop(0, BLK // L, inner, 0, unroll=4)
            return 0

        lax.fori_loop(0, NBLK, blk, 0)

    tasks = ((iarr, tf), (jarr, tb), (iarr, None), (jarr, None))
    for kk, (idx_ref, val_ref) in enumerate(tasks):
        @pl.when(kind == kk)
        def _(idx_ref=idx_ref, val_ref=val_ref):
            scatter_task(idx_ref, val_ref, chunk)

    pltpu.sync_copy(acc, part_out.at[pl.ds(wid * NN, NN)])


_scatter_call = pl.kernel(
    _sc_scatter_body,
    out_type=jax.ShapeDtypeStruct((NW * NN,), jnp.float32),
    mesh=_mesh,
    compiler_params=pltpu.CompilerParams(needs_layout_passes=False),
    scratch_types=[
        pltpu.VMEM((NN,), jnp.float32),
        pltpu.VMEM((BLK,), jnp.int32),
        pltpu.VMEM((BLK,), jnp.float32),
    ],
)


def _combine_body(p_ref, s_ref):
    p = p_ref[...]
    sf = jnp.sum(p[0:8], axis=0, keepdims=True)
    sb = jnp.sum(p[8:16], axis=0, keepdims=True)
    ci = jnp.sum(p[16:24], axis=0, keepdims=True)
    cj = jnp.sum(p[24:32], axis=0, keepdims=True)
    s_ref[...] = sf / jnp.maximum(ci, 1.0) + sb / jnp.maximum(cj, 1.0)


def _combine_call(P):
    return pl.pallas_call(
        _combine_body,
        grid=(pl.cdiv(NN, DB),),
        in_specs=[pl.BlockSpec((NW, DB), lambda b: (0, b))],
        out_specs=pl.BlockSpec((1, DB), lambda b: (0, b)),
        out_shape=jax.ShapeDtypeStruct((1, NN), jnp.float32),
    )(P)


def kernel(edge_index, r_ij, v, W1, b1, W2, b2, W3, b3):
    ei = edge_index.astype(jnp.int32)
    iarr = ei[0]
    jarr = ei[1]
    rflat = r_ij.reshape(-1)
    v0, v1, v2 = v[:, 0], v[:, 1], v[:, 2]
    r2, d0, d1, d2 = _gather_call(iarr, jarr, rflat, v0, v1, v2)
    tf, tb = _mlp_call(r2.reshape(1, E), d0.reshape(1, E), d1.reshape(1, E),
                       d2.reshape(1, E), W1, b1.reshape(32, 1), W2,
                       b2.reshape(32, 1), W3.reshape(32, 1), b3.reshape(1, 1))
    P = _scatter_call(iarr, jarr, tf.reshape(E), tb.reshape(E))
    S = _combine_call(P.reshape(NW, NN))
    return S.reshape(NN, 1)


# speedup vs baseline: 6.7768x; 1.0041x over previous
import jax
import jax.numpy as jnp
from jax import lax
from jax.experimental import pallas as pl
from jax.experimental.pallas import tpu as pltpu
from jax.experimental.pallas import tpu_sc as plsc

E = 6_400_000
NN = 100_000
L = 16
NW = 32
NCHUNK = 8
CHUNK = E // NCHUNK
BLK = 8_000
NBLK = CHUNK // BLK
TCB = 10_240
DB = 8_192

_mesh = plsc.VectorSubcoreMesh(core_axis_name="c", subcore_axis_name="s")


def _sc_gather_body(eiflat, rflat, vtflat,
                    r2_out, d0_out, d1_out, d2_out,
                    tab, ibuf, jbuf, obuf):
    wid = lax.axis_index("s") * 2 + lax.axis_index("c")

    def v_task(comp, dout, chunk):
        pltpu.sync_copy(vtflat.at[pl.ds(comp * NN, NN)], tab)

        def blk(b, _):
            base = chunk * CHUNK + b * BLK
            pltpu.sync_copy(eiflat.at[pl.ds(base, BLK)], ibuf)
            pltpu.sync_copy(eiflat.at[pl.ds(E + base, BLK)], jbuf)

            def inner(k, _):
                sl = pl.ds(k * L, L)
                gi = plsc.load_gather(tab, [ibuf[sl]])
                gj = plsc.load_gather(tab, [jbuf[sl]])
                obuf[sl] = gi - gj
                return 0

            lax.fori_loop(0, BLK // L, inner, 0, unroll=4)
            pltpu.sync_copy(obuf, dout.at[pl.ds(base, BLK)])
            return 0

        lax.fori_loop(0, NBLK, blk, 0)

    for c, dout in enumerate((d0_out, d1_out, d2_out)):
        @pl.when(jnp.logical_and(wid >= c * NCHUNK, wid < (c + 1) * NCHUNK))
        def _(dout=dout, c=c):
            v_task(c, dout, wid - c * NCHUNK)

    @pl.when(wid >= 24)
    def _r_task():
        chunk = wid - 24
        sidx = lax.iota(jnp.int32, L) * 3

        def blk(b, _):
            base = chunk * CHUNK + b * BLK
            pltpu.sync_copy(rflat.at[pl.ds(base * 3, BLK * 3)],
                            tab.at[pl.ds(0, BLK * 3)])

            def inner(k, _):
                off = sidx + k * (3 * L)
                x = plsc.load_gather(tab, [off])
                y = plsc.load_gather(tab, [off + 1])
                z = plsc.load_gather(tab, [off + 2])
                obuf[pl.ds(k * L, L)] = x * x + y * y + z * z
                return 0

            lax.fori_loop(0, BLK // L, inner, 0, unroll=4)
            pltpu.sync_copy(obuf, r2_out.at[pl.ds(base, BLK)])
            return 0

        lax.fori_loop(0, NBLK, blk, 0)


_gather_call = pl.kernel(
    _sc_gather_body,
    out_type=tuple(jax.ShapeDtypeStruct((E,), jnp.float32) for _ in range(4)),
    mesh=_mesh,
    compiler_params=pltpu.CompilerParams(needs_layout_passes=False),
    scratch_types=[
        pltpu.VMEM((NN,), jnp.float32),
        pltpu.VMEM((BLK,), jnp.int32),
        pltpu.VMEM((BLK,), jnp.int32),
        pltpu.VMEM((BLK,), jnp.float32),
    ],
)


def _silu(x):
    return x * (1.0 / (1.0 + jnp.exp(-x)))


def _mlp_body(r2_ref, d0_ref, d1_ref, d2_ref,
              w1_ref, b1_ref, w2_ref, b2_ref, w3_ref, b3_ref,
              tf_ref, tb_ref):
    r = jnp.sqrt(r2_ref[...]).reshape(1, TCB)
    w1 = w1_ref[...]
    a = w1[:, 0:1] * r + b1_ref[...]
    bb = (w1[:, 1:2] * d0_ref[...].reshape(1, TCB)
          + w1[:, 2:3] * d1_ref[...].reshape(1, TCB)
          + w1[:, 3:4] * d2_ref[...].reshape(1, TCB))
    h1f = _silu(a + bb)
    h1b = _silu(a - bb)
    w2 = w2_ref[...]
    h2f = _silu(jnp.dot(w2, h1f, preferred_element_type=jnp.float32) + b2_ref[...])
    h2b = _silu(jnp.dot(w2, h1b, preferred_element_type=jnp.float32) + b2_ref[...])
    w3c = w3_ref[...]
    tf_ref[...] = (jnp.sum(w3c * h2f, axis=0, keepdims=True)
                   + b3_ref[...]).reshape(TCB)
    tb_ref[...] = (jnp.sum(w3c * h2b, axis=0, keepdims=True)
                   + b3_ref[...]).reshape(TCB)


def _mlp_call(r2, d0, d1, d2, W1, b1c, W2, b2c, w3c, b3c):
    edge_spec = pl.BlockSpec((TCB,), lambda b: (b,))
    return pl.pallas_call(
        _mlp_body,
        grid=(E // TCB,),
        in_specs=[
            edge_spec, edge_spec, edge_spec, edge_spec,
            pl.BlockSpec((32, 4), lambda b: (0, 0)),
            pl.BlockSpec((32, 1), lambda b: (0, 0)),
            pl.BlockSpec((32, 32), lambda b: (0, 0)),
            pl.BlockSpec((32, 1), lambda b: (0, 0)),
            pl.BlockSpec((32, 1), lambda b: (0, 0)),
            pl.BlockSpec((1, 1), lambda b: (0, 0)),
        ],
        out_specs=(edge_spec, edge_spec),
        out_shape=(jax.ShapeDtypeStruct((E,), jnp.float32),
                   jax.ShapeDtypeStruct((E,), jnp.float32)),
    )(r2, d0, d1, d2, W1, b1c, W2, b2c, w3c, b3c)


def _sc_scatter_body(eiflat, tf, tb, part_out, acc, ibuf, vbuf):
    wid = lax.axis_index("s") * 2 + lax.axis_index("c")
    kind = wid // NCHUNK
    chunk = wid % NCHUNK

    def z(k, _):
        acc[pl.ds(k * L, L)] = jnp.zeros((L,), jnp.float32)
        return 0

    lax.fori_loop(0, NN // L, z, 0, unroll=8)
    ones = jnp.ones((L,), jnp.float32)

    def scatter_task(idx_off, val_ref, chunk):
        def blk(b, _):
            base = chunk * CHUNK + b * BLK
            pltpu.sync_copy(eiflat.at[pl.ds(idx_off + base, BLK)], ibuf)
            if val_ref is not None:
                pltpu.sync_copy(val_ref.at[pl.ds(base, BLK)], vbuf)

            def inner(k, _):
                sl = pl.ds(k * L, L)
                val = vbuf[sl] if val_ref is not None else ones
                plsc.addupdate_scatter(acc, [ibuf[sl]], val)
                return 0

            lax.fori_loop(0, BLK // L, inner, 0, unroll=4)
            return 0

        lax.fori_loop(0, NBLK, blk, 0)

    tasks = ((0, tf), (E, tb), (0, None), (E, None))
    for kk, (idx_off, val_ref) in enumerate(tasks):
        @pl.when(kind == kk)
        def _(idx_off=idx_off, val_ref=val_ref):
            scatter_task(idx_off, val_ref, chunk)

    pltpu.sync_copy(acc, part_out.at[pl.ds(wid * NN, NN)])


_scatter_call = pl.kernel(
    _sc_scatter_body,
    out_type=jax.ShapeDtypeStruct((NW * NN,), jnp.float32),
    mesh=_mesh,
    compiler_params=pltpu.CompilerParams(needs_layout_passes=False),
    scratch_types=[
        pltpu.VMEM((NN,), jnp.float32),
        pltpu.VMEM((BLK,), jnp.int32),
        pltpu.VMEM((BLK,), jnp.float32),
    ],
)


def _combine_body(p_ref, s_ref):
    p = p_ref[...]
    sf = jnp.sum(p[0:8], axis=0, keepdims=True)
    sb = jnp.sum(p[8:16], axis=0, keepdims=True)
    ci = jnp.sum(p[16:24], axis=0, keepdims=True)
    cj = jnp.sum(p[24:32], axis=0, keepdims=True)
    s_ref[...] = sf / jnp.maximum(ci, 1.0) + sb / jnp.maximum(cj, 1.0)


def _combine_call(P):
    return pl.pallas_call(
        _combine_body,
        grid=(pl.cdiv(NN, DB),),
        in_specs=[pl.BlockSpec((NW, DB), lambda b: (0, b))],
        out_specs=pl.BlockSpec((1, DB), lambda b: (0, b)),
        out_shape=jax.ShapeDtypeStruct((1, NN), jnp.float32),
    )(P)


def kernel(edge_index, r_ij, v, W1, b1, W2, b2, W3, b3):
    eiflat = edge_index.astype(jnp.int32).reshape(2 * E)
    rflat = r_ij.reshape(3 * E)
    vtflat = v.T.reshape(3 * NN)
    r2, d0, d1, d2 = _gather_call(eiflat, rflat, vtflat)
    tf, tb = _mlp_call(r2, d0, d1, d2, W1, b1.reshape(32, 1), W2,
                       b2.reshape(32, 1), W3.reshape(32, 1), b3.reshape(1, 1))
    P = _scatter_call(eiflat, tf, tb)
    S = _combine_call(P.reshape(NW, NN))
    return S.reshape(NN, 1)
